# baseline (speedup 1.0000x reference)
import functools

import jax
import jax.numpy as jnp
from jax import lax
from jax.experimental import pallas as pl
from jax.experimental.pallas import tpu as pltpu
from jax.experimental.pallas import tpu_sc as plsc

N = 2048
H = 128
W = 128
M0 = 0.01
BOUND = M0 * N
K = 21
CH = 128
NR = 8
RS = 96
SC_ROWS = H - RS


def _dtm_tc_kernel(x_ref, out_ref, *scratch):
    krefs, dx2_ref = scratch[:-1], scratch[-1]
    MININT = jnp.int32(-2147483648)
    CSHIFT = jnp.int32(1) - MININT
    i = pl.program_id(0)

    gx = -1.0 + jax.lax.broadcasted_iota(
        jnp.int32, (1, W), 1).astype(jnp.float32) * (2.0 / (W - 1))
    px = x_ref[:, 0:1]
    py = x_ref[:, 1:2]

    @pl.when(i == 0)
    def _():
        dxv = px - gx
        dx2_ref[...] = dxv * dxv

    dx2 = dx2_ref[...]
    row = jax.lax.broadcasted_iota(jnp.int32, (N, 1), 0)
    mask_hi = jnp.int32(~2047)

    for r, kref in enumerate(krefs):
        gy = 1.0 - (NR * i + r).astype(jnp.float32) * (2.0 / (W - 1))
        dy = py - gy
        d2 = dx2 + dy * dy
        bits = jax.lax.bitcast_convert_type(d2, jnp.int32)
        kref[...] = (bits & mask_hi) | row

    def masked_min(kref, shift):
        acc = None
        for c in range(0, N, CH):
            e = kref[c:c + CH, :] - shift
            while e.shape[0] > 8:
                h = e.shape[0] // 2
                e = jnp.minimum(e[:h], e[h:])
            acc = e if acc is None else jnp.minimum(acc, e)
        while acc.shape[0] > 1:
            h = acc.shape[0] // 2
            acc = jnp.minimum(acc[:h], acc[h:])
        return acc

    def body(_, carry):
        ss, vs = carry
        ms = [masked_min(kref, v + CSHIFT) for kref, v in zip(krefs, vs)]
        vs = tuple(v + jnp.int32(1) + (m ^ MININT) for v, m in zip(vs, ms))
        ss = tuple(s + jax.lax.bitcast_convert_type(v, jnp.float32)
                   for s, v in zip(ss, vs))
        return ss, vs

    zf = jnp.zeros((1, W), jnp.float32)
    zi = jnp.full((1, W), -1, jnp.int32)
    ss, ts = jax.lax.fori_loop(0, K, body, ((zf,) * NR, (zi,) * NR))

    for r in range(NR):
        tf = jax.lax.bitcast_convert_type(ts[r], jnp.float32)
        out_ref[0, r:r + 1, :] = jnp.sqrt((ss[r] + tf * (BOUND - K)) / BOUND)


def _tc_part(x):
    out = pl.pallas_call(
        _dtm_tc_kernel,
        grid=(RS // NR,),
        in_specs=[pl.BlockSpec((N, 2), lambda i: (0, 0))],
        out_specs=pl.BlockSpec((1, NR, W), lambda i: (i, 0, 0)),
        out_shape=jax.ShapeDtypeStruct((RS // NR, NR, W), jnp.float32),
        scratch_shapes=[pltpu.VMEM((N, W), jnp.int32)] * NR
                       + [pltpu.VMEM((N, W), jnp.float32)],
    )(x)
    return out.reshape(RS, W)


def _keygen_kernel(xt_ref, kout_ref):
    i = pl.program_id(0)
    gxt = -1.0 + jax.lax.broadcasted_iota(
        jnp.int32, (W, 1), 0).astype(jnp.float32) * (2.0 / (W - 1))
    px = xt_ref[0:1, :]
    py = xt_ref[1:2, :]
    gy = 1.0 - (RS + i).astype(jnp.float32) * (2.0 / (W - 1))
    dx = gxt - px
    dy = gy - py
    d2 = dx * dx + dy * dy
    col = jax.lax.broadcasted_iota(jnp.int32, (1, N), 1)
    bits = jax.lax.bitcast_convert_type(d2, jnp.int32)
    kout_ref[0] = (bits & jnp.int32(~2047)) | col


def _keygen(x):
    return pl.pallas_call(
        _keygen_kernel,
        grid=(SC_ROWS,),
        in_specs=[pl.BlockSpec((2, N), lambda i: (0, 0))],
        out_specs=pl.BlockSpec((1, W, N), lambda i: (i, 0, 0)),
        out_shape=jax.ShapeDtypeStruct((SC_ROWS, W, N), jnp.int32),
    )(x.T)


def _sc_part(x):
    mesh = plsc.VectorSubcoreMesh(core_axis_name="c", subcore_axis_name="s",
                                  num_cores=2, num_subcores=16)

    @functools.partial(
        pl.kernel, mesh=mesh,
        out_type=jax.ShapeDtypeStruct((SC_ROWS, W), jnp.float32),
        scratch_types=[
            pltpu.VMEM((N // 8, 128), jnp.int32),
            pltpu.VMEM((W,), jnp.float32),
        ],
    )
    def sc_kernel(keys_hbm, out_hbm, keyv, outv):
        MININT = jnp.int32(-2147483648)
        CSHIFT = jnp.int32(1) - MININT
        wid = lax.axis_index("s") * 2 + lax.axis_index("c")

        for c in range(W // 16):
            pltpu.sync_copy(keys_hbm.at[wid * (W // 16) + c], keyv)

            def ext_body(_, carry):
                s, v = carry
                shift = v + CSHIFT

                def scan_body(r, accs):
                    return tuple(
                        jnp.minimum(a, keyv[r, 16 * g:16 * (g + 1)] - shift)
                        for g, a in enumerate(accs))

                big = jnp.full((16,), 2147483647, jnp.int32)
                accs = lax.fori_loop(
                    0, N // 8, scan_body, (big,) * 8, unroll=2)
                while len(accs) > 1:
                    accs = tuple(jnp.minimum(accs[2 * j], accs[2 * j + 1])
                                 for j in range(len(accs) // 2))
                m = accs[0]
                v = v + jnp.int32(1) + (m ^ MININT)
                s = s + jax.lax.bitcast_convert_type(v, jnp.float32)
                return s, v

            s, t = lax.fori_loop(
                0, K, ext_body,
                (jnp.zeros((16,), jnp.float32), jnp.full((16,), -1, jnp.int32)))

            tf = jax.lax.bitcast_convert_type(t, jnp.float32)
            outv[c * 16:(c + 1) * 16] = (s + tf * (BOUND - K)) / BOUND

        pltpu.sync_copy(outv, out_hbm.at[wid])

    keys = _keygen(x).reshape(SC_ROWS, W // 16, 16, N)
    keys = keys.transpose(0, 1, 3, 2).reshape(SC_ROWS * (W // 16), N // 8, 128)
    return jnp.sqrt(sc_kernel(keys))


@jax.jit
def kernel(x):
    return jnp.concatenate([_tc_part(x), _sc_part(x)], axis=0)
